# Initial kernel scaffold; baseline (speedup 1.0000x reference)
import jax
import jax.numpy as jnp
from jax.experimental import pallas as pl


def kernel(x, pos_embedding, scale):
    raise NotImplementedError("write your pallas kernel here")


# speedup vs baseline: 1.4920x; 1.4920x over previous
import jax
import jax.numpy as jnp
from jax.experimental import pallas as pl
from jax.experimental.pallas import tpu as pltpu

_S_CHUNK = 512


def _add_pos_kernel(x_ref, pos_ref, scale_ref, out_ref):
    out_ref[...] = x_ref[...] + pos_ref[...] * scale_ref[0]


def kernel(x, pos_embedding, scale):
    batch, seq_len, dim = x.shape
    num_chunks = seq_len // _S_CHUNK
    pos = pos_embedding[:seq_len]
    return pl.pallas_call(
        _add_pos_kernel,
        grid=(num_chunks, batch),
        in_specs=[
            pl.BlockSpec((1, _S_CHUNK, dim), lambda i, j: (j, i, 0)),
            pl.BlockSpec((_S_CHUNK, dim), lambda i, j: (i, 0)),
            pl.BlockSpec(memory_space=pltpu.SMEM),
        ],
        out_specs=pl.BlockSpec((1, _S_CHUNK, dim), lambda i, j: (j, i, 0)),
        out_shape=jax.ShapeDtypeStruct(x.shape, x.dtype),
    )(x, pos, scale)


# speedup vs baseline: 1.6608x; 1.1132x over previous
import jax
import jax.numpy as jnp
from jax.experimental import pallas as pl
from jax.experimental.pallas import tpu as pltpu

_S_CHUNK = 1024


def _add_pos_kernel(x_ref, pos_ref, scale_ref, out_ref):
    out_ref[...] = x_ref[...] + pos_ref[...] * scale_ref[0]


def kernel(x, pos_embedding, scale):
    batch, seq_len, dim = x.shape
    num_chunks = seq_len // _S_CHUNK
    pos = pos_embedding[:seq_len]
    return pl.pallas_call(
        _add_pos_kernel,
        grid=(num_chunks, batch),
        in_specs=[
            pl.BlockSpec((1, _S_CHUNK, dim), lambda i, j: (j, i, 0)),
            pl.BlockSpec((_S_CHUNK, dim), lambda i, j: (i, 0)),
            pl.BlockSpec(memory_space=pltpu.SMEM),
        ],
        out_specs=pl.BlockSpec((1, _S_CHUNK, dim), lambda i, j: (j, i, 0)),
        out_shape=jax.ShapeDtypeStruct(x.shape, x.dtype),
    )(x, pos, scale)


# speedup vs baseline: 1.7275x; 1.0402x over previous
import jax
import jax.numpy as jnp
from jax.experimental import pallas as pl
from jax.experimental.pallas import tpu as pltpu

_S_CHUNK = 2048


def _add_pos_kernel(x_ref, pos_ref, scale_ref, out_ref):
    out_ref[...] = x_ref[...] + pos_ref[...] * scale_ref[0]


def kernel(x, pos_embedding, scale):
    batch, seq_len, dim = x.shape
    num_chunks = seq_len // _S_CHUNK
    pos = pos_embedding[:seq_len]
    return pl.pallas_call(
        _add_pos_kernel,
        grid=(num_chunks, batch),
        in_specs=[
            pl.BlockSpec((1, _S_CHUNK, dim), lambda i, j: (j, i, 0)),
            pl.BlockSpec((_S_CHUNK, dim), lambda i, j: (i, 0)),
            pl.BlockSpec(memory_space=pltpu.SMEM),
        ],
        out_specs=pl.BlockSpec((1, _S_CHUNK, dim), lambda i, j: (j, i, 0)),
        out_shape=jax.ShapeDtypeStruct(x.shape, x.dtype),
    )(x, pos, scale)


# speedup vs baseline: 1.7291x; 1.0009x over previous
import jax
import jax.numpy as jnp
from jax.experimental import pallas as pl
from jax.experimental.pallas import tpu as pltpu

_S_CHUNK = 2048


def _add_pos_kernel(x_ref, pos_ref, scale_ref, out_ref):
    out_ref[...] = x_ref[...] + pos_ref[...] * scale_ref[0]


def kernel(x, pos_embedding, scale):
    batch, seq_len, dim = x.shape
    num_chunks = seq_len // _S_CHUNK
    pos = pos_embedding[:seq_len]
    return pl.pallas_call(
        _add_pos_kernel,
        grid=(num_chunks, batch),
        in_specs=[
            pl.BlockSpec((1, _S_CHUNK, dim), lambda i, j: (j, i, 0)),
            pl.BlockSpec((_S_CHUNK, dim), lambda i, j: (i, 0)),
            pl.BlockSpec(memory_space=pltpu.SMEM),
        ],
        out_specs=pl.BlockSpec((1, _S_CHUNK, dim), lambda i, j: (j, i, 0)),
        out_shape=jax.ShapeDtypeStruct(x.shape, x.dtype),
        compiler_params=pltpu.CompilerParams(
            vmem_limit_bytes=112 * 1024 * 1024,
        ),
    )(x, pos, scale)
